# baseline (speedup 1.0000x reference)
import jax
import jax.numpy as jnp
from jax import lax
from jax.experimental import pallas as pl
from jax.experimental.pallas import tpu as pltpu
from jax.experimental.pallas import tpu_sc as plsc

_NC = 2
_NS = 16
_NW = _NC * _NS
_D = 16
_F = 128
_C = 8
_NBUF = 4
_SC_PARAMS = pltpu.CompilerParams(needs_layout_passes=False)

_GATHER_DNUMS = lax.GatherDimensionNumbers(
    offset_dims=(), collapsed_slice_dims=(0,), start_index_map=(0,))


def _lane_gather(vec, idx):
    return lax.gather(vec, idx[:, None], _GATHER_DNUMS, slice_sizes=(1,),
                      mode=lax.GatherScatterMode.PROMISE_IN_BOUNDS)


def _make_sc_compose(n_rows_pad, bw, n_ids):
    mesh = plsc.VectorSubcoreMesh(core_axis_name="c", subcore_axis_name="s")
    bwp = -(-bw // 128) * 128
    scratch = [
        pltpu.VMEM((n_ids,), jnp.int32),
        pltpu.VMEM((bw * _D,), jnp.int32),
        pltpu.VMEM((bwp,), jnp.int32),
    ]
    out_type = (
        jax.ShapeDtypeStruct((n_rows_pad * _D,), jnp.int32),
        jax.ShapeDtypeStruct((n_rows_pad,), jnp.int32),
    )

    def body(nidx, selfpos, ids_hbm, cidx_out, cself_out,
             ids_v, cidx_v, cself_v):
        wid = lax.axis_index("s") * _NC + lax.axis_index("c")
        base = wid * bw
        pltpu.sync_copy(ids_hbm, ids_v)
        pltpu.sync_copy(nidx.at[pl.ds(base * _D, bw * _D)], cidx_v)
        pltpu.sync_copy(selfpos.at[pl.ds(base, bw)], cself_v.at[pl.ds(0, bw)])

        def comp_n(i, carry):
            v = cidx_v[pl.ds(i * 16, 16)]
            cidx_v[pl.ds(i * 16, 16)] = plsc.load_gather(ids_v, [v])
            return carry

        lax.fori_loop(0, bw * _D // 16, comp_n, 0)

        def comp_s(i, carry):
            v = cself_v[pl.ds(i * 16, 16)]
            cself_v[pl.ds(i * 16, 16)] = plsc.load_gather(ids_v, [v])
            return carry

        lax.fori_loop(0, bw // 16, comp_s, 0)
        pltpu.sync_copy(cidx_v, cidx_out.at[pl.ds(base * _D, bw * _D)])
        pltpu.sync_copy(cself_v.at[pl.ds(0, bw)],
                        cself_out.at[pl.ds(base, bw)])

    return pl.kernel(body, out_type=out_type, mesh=mesh,
                     scratch_types=scratch, compiler_params=_SC_PARAMS,
                     name="sc_compose")


def _make_sc_pool(n_rows_pad, bw0, bw1, packed=False):
    nchunk0 = bw0 // _C
    nchunk1 = bw1 // _C
    bwm = max(bw0, bw1)
    assert min(nchunk0, nchunk1) >= _NBUF
    mesh = plsc.VectorSubcoreMesh(core_axis_name="c", subcore_axis_name="s")
    scratch = [
        pltpu.VMEM((bwm * _D,), jnp.int32),
        pltpu.VMEM((-(-bwm // 128) * 128,), jnp.int32),
        pltpu.VMEM((bwm * _D,), jnp.float32),
        pltpu.VMEM((_NBUF, _C * _D, _F // 2), jnp.int32) if packed
        else pltpu.VMEM((_NBUF, _C * _D, _F), jnp.float32),
        pltpu.VMEM((_NBUF, _C, _F), jnp.float32),
        pltpu.VMEM((_NBUF, _C, _F), jnp.float32),
        pltpu.SemaphoreType.DMA((_NBUF,)),
        pltpu.SemaphoreType.DMA((_NBUF,)),
    ]
    out_type = (
        jax.ShapeDtypeStruct((n_rows_pad, _F), jnp.float32),
        jax.ShapeDtypeStruct((n_rows_pad, _F), jnp.float32),
    )
    params = pltpu.CompilerParams(needs_layout_passes=False,
                                  use_tc_tiling_on_sc=False)

    def body(table_n, table_s, nidx, selfpos, w, self_out, mean_out,
             cidx_v, cself_v, w_v, rows_v, srows_v, nm_v, isem, osem):
        c = lax.axis_index("c")
        s = lax.axis_index("s")
        is0 = c == 0
        base = lax.select(is0, s * bw0, _NS * bw0 + s * bw1)
        nchunk = lax.select(is0, nchunk0, nchunk1)

        pltpu.sync_copy(nidx.at[pl.ds(base * _D, bwm * _D)], cidx_v)
        pltpu.sync_copy(selfpos.at[pl.ds(base, bwm)],
                        cself_v.at[pl.ds(0, bwm)])
        pltpu.sync_copy(w.at[pl.ds(base * _D, bwm * _D)], w_v)

        def start_fetch(c, buf):
            pltpu.async_copy(
                table_n.at[cidx_v.at[pl.ds(c * (_C * _D), _C * _D)]],
                rows_v.at[buf], isem.at[buf])
            pltpu.async_copy(
                table_s.at[cself_v.at[pl.ds(c * _C, _C)]],
                srows_v.at[buf], isem.at[buf])

        def wait_outputs(c, buf):
            pltpu.make_async_copy(
                srows_v.at[buf],
                self_out.at[pl.ds(base + (c - _NBUF) * _C, _C)],
                osem.at[buf]).wait()
            pltpu.make_async_copy(
                nm_v.at[buf],
                mean_out.at[pl.ds(base + (c - _NBUF) * _C, _C)],
                osem.at[buf]).wait()

        for b in range(_NBUF - 1):
            start_fetch(b, b)

        def chunk(c, carry):
            buf = lax.rem(c, _NBUF)
            nxt = c + _NBUF - 1

            @pl.when(nxt < nchunk)
            def _():
                nbuf = lax.rem(nxt, _NBUF)

                @pl.when(nxt >= _NBUF)
                def _():
                    wait_outputs(nxt, nbuf)

                start_fetch(nxt, nbuf)

            pltpu.make_async_copy(
                table_n.at[cidx_v.at[pl.ds(c * (_C * _D), _C * _D)]],
                rows_v.at[buf], isem.at[buf]).wait()
            pltpu.make_async_copy(
                table_s.at[cself_v.at[pl.ds(c * _C, _C)]],
                srows_v.at[buf], isem.at[buf]).wait()
            rb = rows_v.at[buf]
            nb = nm_v.at[buf]
            for b in range(_C):
                wv = w_v[pl.ds(c * (_C * _D) + b * _D, _D)]
                tot = _lane_gather(plsc.cumsum(wv),
                                   jnp.full((16,), _D - 1, jnp.int32))
                r = 1.0 / (tot + 1e-8)
                acc = [None] * (_F // 16)
                for j in range(_D):
                    wj = _lane_gather(wv, jnp.full((16,), j, jnp.int32))
                    if packed:
                        for k in range(_F // 32):
                            wd = rb[b * _D + j, pl.ds(k * 16, 16)]
                            lo = lax.bitcast_convert_type(
                                wd << 16, jnp.float32)
                            hi = lax.bitcast_convert_type(
                                wd & jnp.int32(-65536), jnp.float32)
                            xl = wj * lo
                            xh = wj * hi
                            kk = _F // 32 + k
                            acc[k] = xl if acc[k] is None else acc[k] + xl
                            acc[kk] = xh if acc[kk] is None else acc[kk] + xh
                    else:
                        for k in range(_F // 16):
                            x = wj * rb[b * _D + j, pl.ds(k * 16, 16)]
                            acc[k] = x if acc[k] is None else acc[k] + x
                for k in range(_F // 16):
                    nb[b, pl.ds(k * 16, 16)] = acc[k] * r
            pltpu.async_copy(srows_v.at[buf],
                             self_out.at[pl.ds(base + c * _C, _C)],
                             osem.at[buf])
            pltpu.async_copy(nb, mean_out.at[pl.ds(base + c * _C, _C)],
                             osem.at[buf])
            return carry

        lax.fori_loop(0, nchunk, chunk, 0, unroll=False)
        for b in range(_NBUF):
            wait_outputs(nchunk + b, lax.rem(nchunk + b, _NBUF))

    return pl.kernel(body, out_type=out_type, mesh=mesh,
                     scratch_types=scratch, compiler_params=params,
                     name="sc_pool")


def _tc_mlp_body(self_ref, nm_ref, wa_ref, ba_ref, ws_ref, wg_ref, be_ref,
                 out_ref):
    agg = jnp.dot(nm_ref[...], wa_ref[...],
                  preferred_element_type=jnp.float32) + ba_ref[...]
    agg = jnp.maximum(agg, 0.0)
    h = jnp.dot(self_ref[...], ws_ref[...],
                preferred_element_type=jnp.float32)
    h = h + jnp.dot(agg, wg_ref[...], preferred_element_type=jnp.float32)
    h = jnp.maximum(h + be_ref[...], 0.0)
    n = jnp.sqrt(jnp.sum(h * h, axis=1, keepdims=True)) + 1e-8
    out_ref[...] = h / n


def _tc_mlp(self_f, nm, Wa, ba, We, be, blk=256):
    n = self_f.shape[0]
    grid = (n // blk,)
    row_spec = pl.BlockSpec((blk, _F), lambda i: (i, 0))
    w_spec = pl.BlockSpec((_F, _F), lambda i: (0, 0))
    b_spec = pl.BlockSpec((1, _F), lambda i: (0, 0))
    return pl.pallas_call(
        _tc_mlp_body,
        grid=grid,
        in_specs=[row_spec, row_spec, w_spec, b_spec, w_spec, w_spec, b_spec],
        out_specs=row_spec,
        out_shape=jax.ShapeDtypeStruct((n, _F), jnp.float32),
    )(self_f, nm, Wa, ba.reshape(1, _F), We[:_F], We[_F:], be.reshape(1, _F))


def _pad_rows(x, n_pad):
    pad = [(0, n_pad - x.shape[0])] + [(0, 0)] * (x.ndim - 1)
    return jnp.pad(x, pad)


@jax.jit
def kernel(node_ids_l0, nodes_l1_pos, nodes_l2_pos, neigh_idx_l0, neigh_w_l0,
           neigh_idx_l1, neigh_w_l1, embedding_table,
           W_agg1, b_agg1, W_enc1, b_enc1, W_agg2, b_agg2, W_enc2, b_enc2):
    i32 = jnp.int32
    n1 = nodes_l1_pos.shape[0]
    n2 = nodes_l2_pos.shape[0]
    bwc = -(-n1 // (_NW * _C)) * _C
    n1p = bwc * _NW
    pair1 = n1p // _NS
    bw0_1 = max(_NBUF * _C, int(pair1 * 0.74) // _C * _C)
    bw1_1 = pair1 - bw0_1
    stage1 = _NS * bw0_1 + (_NS - 1) * bw1_1 + max(bw0_1, bw1_1)
    bw_2 = -(-n2 // (_NW * _C)) * _C
    n2p = bw_2 * _NW
    n0p = -(-node_ids_l0.shape[0] // 128) * 128

    nidx1 = _pad_rows(neigh_idx_l0.astype(i32), n1p).reshape(-1)
    w1 = _pad_rows(neigh_w_l0, n1p).reshape(-1)
    sp1 = _pad_rows(nodes_l1_pos.astype(i32), n1p)
    cidx1, cself1 = _make_sc_compose(n1p, bwc, n0p)(
        nidx1, sp1, _pad_rows(node_ids_l0.astype(i32), n0p))
    v = embedding_table.shape[0]
    packed1 = lax.bitcast_convert_type(
        embedding_table.astype(jnp.bfloat16).reshape(v, _F // 2, 2),
        jnp.int32)
    self1, mean1 = _make_sc_pool(n1p, bw0_1, bw1_1, packed=True)(
        packed1, embedding_table,
        jnp.pad(cidx1, (0, (stage1 - n1p) * _D)),
        jnp.pad(cself1, (0, stage1 - n1p)),
        jnp.pad(w1, (0, (stage1 - n1p) * _D)))
    perm = jnp.concatenate([jnp.arange(0, _F, 2), jnp.arange(1, _F, 2)])
    h1 = _tc_mlp(self1, mean1, W_agg1[perm], b_agg1, W_enc1, b_enc1)

    nidx2 = _pad_rows(neigh_idx_l1.astype(i32), n2p).reshape(-1)
    w2 = _pad_rows(neigh_w_l1, n2p).reshape(-1)
    sp2 = _pad_rows(nodes_l2_pos.astype(i32), n2p)
    self2, mean2 = _make_sc_pool(n2p, bw_2, bw_2)(h1, h1, nidx2, sp2, w2)
    h2 = _tc_mlp(self2, mean2, W_agg2, b_agg2, W_enc2, b_enc2)
    return h2[:n2]


# speedup vs baseline: 1.7081x; 1.7081x over previous
import jax
import jax.numpy as jnp
from jax import lax
from jax.experimental import pallas as pl
from jax.experimental.pallas import tpu as pltpu
from jax.experimental.pallas import tpu_sc as plsc

_NC = 2
_NS = 16
_NW = _NC * _NS
_D = 16
_F = 128
_C = 8
_NBUF = 4
_SC_PARAMS = pltpu.CompilerParams(needs_layout_passes=False)

_GATHER_DNUMS = lax.GatherDimensionNumbers(
    offset_dims=(), collapsed_slice_dims=(0,), start_index_map=(0,))


def _lane_gather(vec, idx):
    return lax.gather(vec, idx[:, None], _GATHER_DNUMS, slice_sizes=(1,),
                      mode=lax.GatherScatterMode.PROMISE_IN_BOUNDS)


def _make_sc_compose(n_rows_pad, bw, n_ids):
    mesh = plsc.VectorSubcoreMesh(core_axis_name="c", subcore_axis_name="s")
    bwp = -(-bw // 128) * 128
    scratch = [
        pltpu.VMEM((n_ids,), jnp.int32),
        pltpu.VMEM((bw * _D,), jnp.int32),
        pltpu.VMEM((bwp,), jnp.int32),
    ]
    out_type = (
        jax.ShapeDtypeStruct((n_rows_pad * _D,), jnp.int32),
        jax.ShapeDtypeStruct((n_rows_pad,), jnp.int32),
    )

    def body(nidx, selfpos, ids_hbm, cidx_out, cself_out,
             ids_v, cidx_v, cself_v):
        wid = lax.axis_index("s") * _NC + lax.axis_index("c")
        base = wid * bw
        pltpu.sync_copy(ids_hbm, ids_v)
        pltpu.sync_copy(nidx.at[pl.ds(base * _D, bw * _D)], cidx_v)
        pltpu.sync_copy(selfpos.at[pl.ds(base, bw)], cself_v.at[pl.ds(0, bw)])

        def comp_n(i, carry):
            v = cidx_v[pl.ds(i * 16, 16)]
            cidx_v[pl.ds(i * 16, 16)] = plsc.load_gather(ids_v, [v])
            return carry

        lax.fori_loop(0, bw * _D // 16, comp_n, 0)

        def comp_s(i, carry):
            v = cself_v[pl.ds(i * 16, 16)]
            cself_v[pl.ds(i * 16, 16)] = plsc.load_gather(ids_v, [v])
            return carry

        lax.fori_loop(0, bw // 16, comp_s, 0)
        pltpu.sync_copy(cidx_v, cidx_out.at[pl.ds(base * _D, bw * _D)])
        pltpu.sync_copy(cself_v.at[pl.ds(0, bw)],
                        cself_out.at[pl.ds(base, bw)])

    return pl.kernel(body, out_type=out_type, mesh=mesh,
                     scratch_types=scratch, compiler_params=_SC_PARAMS,
                     name="sc_compose")


def _make_sc_pool(n_rows_pad, bw0, bw1, packed=False):
    nchunk0 = bw0 // _C
    nchunk1 = bw1 // _C
    bwm = max(bw0, bw1)
    assert min(nchunk0, nchunk1) >= _NBUF
    mesh = plsc.VectorSubcoreMesh(core_axis_name="c", subcore_axis_name="s")
    scratch = [
        pltpu.VMEM((bwm * _D,), jnp.int32),
        pltpu.VMEM((-(-bwm // 128) * 128,), jnp.int32),
        pltpu.VMEM((bwm * _D,), jnp.float32),
        pltpu.VMEM((_NBUF, _C * _D, _F // 2), jnp.int32) if packed
        else pltpu.VMEM((_NBUF, _C * _D, _F), jnp.float32),
        pltpu.VMEM((_NBUF, _C, _F), jnp.float32),
        pltpu.VMEM((_NBUF, _C, _F), jnp.float32),
        pltpu.SemaphoreType.DMA((_NBUF,)),
        pltpu.SemaphoreType.DMA((_NBUF,)),
    ]
    out_type = (
        jax.ShapeDtypeStruct((n_rows_pad, _F), jnp.float32),
        jax.ShapeDtypeStruct((n_rows_pad, _F), jnp.float32),
    )
    params = pltpu.CompilerParams(needs_layout_passes=False,
                                  use_tc_tiling_on_sc=False)

    def body(table_n, table_s, nidx, selfpos, w, self_out, mean_out,
             cidx_v, cself_v, w_v, rows_v, srows_v, nm_v, isem, osem):
        c = lax.axis_index("c")
        s = lax.axis_index("s")
        is0 = c == 0
        base = lax.select(is0, s * bw0, _NS * bw0 + s * bw1)
        nchunk = lax.select(is0, nchunk0, nchunk1)

        pltpu.sync_copy(nidx.at[pl.ds(base * _D, bwm * _D)], cidx_v)
        pltpu.sync_copy(selfpos.at[pl.ds(base, bwm)],
                        cself_v.at[pl.ds(0, bwm)])
        pltpu.sync_copy(w.at[pl.ds(base * _D, bwm * _D)], w_v)

        def start_fetch(c, buf):
            pltpu.async_copy(
                table_n.at[cidx_v.at[pl.ds(c * (_C * _D), _C * _D)]],
                rows_v.at[buf], isem.at[buf])
            pltpu.async_copy(
                table_s.at[cself_v.at[pl.ds(c * _C, _C)]],
                srows_v.at[buf], isem.at[buf])

        def wait_outputs(c, buf):
            pltpu.make_async_copy(
                srows_v.at[buf],
                self_out.at[pl.ds(base + (c - _NBUF) * _C, _C)],
                osem.at[buf]).wait()
            pltpu.make_async_copy(
                nm_v.at[buf],
                mean_out.at[pl.ds(base + (c - _NBUF) * _C, _C)],
                osem.at[buf]).wait()

        for b in range(_NBUF - 1):
            start_fetch(b, b)

        def chunk(c, carry):
            buf = lax.rem(c, _NBUF)
            nxt = c + _NBUF - 1

            @pl.when(nxt < nchunk)
            def _():
                nbuf = lax.rem(nxt, _NBUF)

                @pl.when(nxt >= _NBUF)
                def _():
                    wait_outputs(nxt, nbuf)

                start_fetch(nxt, nbuf)

            pltpu.make_async_copy(
                table_n.at[cidx_v.at[pl.ds(c * (_C * _D), _C * _D)]],
                rows_v.at[buf], isem.at[buf]).wait()
            pltpu.make_async_copy(
                table_s.at[cself_v.at[pl.ds(c * _C, _C)]],
                srows_v.at[buf], isem.at[buf]).wait()
            rb = rows_v.at[buf]
            nb = nm_v.at[buf]
            for b in range(_C):
                wv = w_v[pl.ds(c * (_C * _D) + b * _D, _D)]
                tot = _lane_gather(plsc.cumsum(wv),
                                   jnp.full((16,), _D - 1, jnp.int32))
                r = 1.0 / (tot + 1e-8)
                acc = [None] * (_F // 16)
                for j in range(_D):
                    wj = _lane_gather(wv, jnp.full((16,), j, jnp.int32))
                    if packed:
                        for k in range(_F // 32):
                            wd = rb[b * _D + j, pl.ds(k * 16, 16)]
                            lo = lax.bitcast_convert_type(
                                wd << 16, jnp.float32)
                            hi = lax.bitcast_convert_type(
                                wd & jnp.int32(-65536), jnp.float32)
                            xl = wj * lo
                            xh = wj * hi
                            kk = _F // 32 + k
                            acc[k] = xl if acc[k] is None else acc[k] + xl
                            acc[kk] = xh if acc[kk] is None else acc[kk] + xh
                    else:
                        for k in range(_F // 16):
                            x = wj * rb[b * _D + j, pl.ds(k * 16, 16)]
                            acc[k] = x if acc[k] is None else acc[k] + x
                for k in range(_F // 16):
                    nb[b, pl.ds(k * 16, 16)] = acc[k] * r
            pltpu.async_copy(srows_v.at[buf],
                             self_out.at[pl.ds(base + c * _C, _C)],
                             osem.at[buf])
            pltpu.async_copy(nb, mean_out.at[pl.ds(base + c * _C, _C)],
                             osem.at[buf])
            return carry

        lax.fori_loop(0, nchunk, chunk, 0, unroll=False)
        for b in range(_NBUF):
            wait_outputs(nchunk + b, lax.rem(nchunk + b, _NBUF))

    return pl.kernel(body, out_type=out_type, mesh=mesh,
                     scratch_types=scratch, compiler_params=params,
                     name="sc_pool")


def _tc_mlp_body(self_ref, nm_ref, wa_ref, ba_ref, ws_ref, wg_ref, be_ref,
                 out_ref):
    agg = jnp.dot(nm_ref[...], wa_ref[...],
                  preferred_element_type=jnp.float32) + ba_ref[...]
    agg = jnp.maximum(agg, 0.0)
    h = jnp.dot(self_ref[...], ws_ref[...],
                preferred_element_type=jnp.float32)
    h = h + jnp.dot(agg, wg_ref[...], preferred_element_type=jnp.float32)
    h = jnp.maximum(h + be_ref[...], 0.0)
    n = jnp.sqrt(jnp.sum(h * h, axis=1, keepdims=True)) + 1e-8
    out_ref[...] = h / n


def _tc_mlp(self_f, nm, Wa, ba, We, be, blk=256):
    n = self_f.shape[0]
    grid = (n // blk,)
    row_spec = pl.BlockSpec((blk, _F), lambda i: (i, 0))
    w_spec = pl.BlockSpec((_F, _F), lambda i: (0, 0))
    b_spec = pl.BlockSpec((1, _F), lambda i: (0, 0))
    return pl.pallas_call(
        _tc_mlp_body,
        grid=grid,
        in_specs=[row_spec, row_spec, w_spec, b_spec, w_spec, w_spec, b_spec],
        out_specs=row_spec,
        out_shape=jax.ShapeDtypeStruct((n, _F), jnp.float32),
    )(self_f, nm, Wa, ba.reshape(1, _F), We[:_F], We[_F:], be.reshape(1, _F))


def _pad_rows(x, n_pad):
    pad = [(0, n_pad - x.shape[0])] + [(0, 0)] * (x.ndim - 1)
    return jnp.pad(x, pad)


@jax.jit
def kernel(node_ids_l0, nodes_l1_pos, nodes_l2_pos, neigh_idx_l0, neigh_w_l0,
           neigh_idx_l1, neigh_w_l1, embedding_table,
           W_agg1, b_agg1, W_enc1, b_enc1, W_agg2, b_agg2, W_enc2, b_enc2):
    i32 = jnp.int32
    n1 = nodes_l1_pos.shape[0]
    n2 = nodes_l2_pos.shape[0]
    bwc = -(-n1 // (_NW * _C)) * _C
    n1p = bwc * _NW
    pair1 = n1p // _NS
    bw0_1 = max(_NBUF * _C, int(pair1 * 0.74) // _C * _C)
    bw1_1 = pair1 - bw0_1
    stage1 = _NS * bw0_1 + (_NS - 1) * bw1_1 + max(bw0_1, bw1_1)
    bw_2 = -(-n2 // (_NW * _C)) * _C
    n2p = bw_2 * _NW
    n0p = -(-node_ids_l0.shape[0] // 128) * 128

    nidx1 = _pad_rows(neigh_idx_l0.astype(i32), n1p).reshape(-1)
    w1 = _pad_rows(neigh_w_l0, n1p).reshape(-1)
    sp1 = _pad_rows(nodes_l1_pos.astype(i32), n1p)
    cidx1, cself1 = _make_sc_compose(n1p, bwc, n0p)(
        nidx1, sp1, _pad_rows(node_ids_l0.astype(i32), n0p))
    tbits = lax.bitcast_convert_type(
        embedding_table.astype(jnp.bfloat16).astype(jnp.float32), jnp.int32)
    packed1 = (((tbits[:, :_F // 2] >> 16) & jnp.int32(0xFFFF))
               | (tbits[:, _F // 2:] & jnp.int32(-65536)))
    self1, mean1 = _make_sc_pool(n1p, bw0_1, bw1_1, packed=True)(
        packed1, embedding_table,
        jnp.pad(cidx1, (0, (stage1 - n1p) * _D)),
        jnp.pad(cself1, (0, stage1 - n1p)),
        jnp.pad(w1, (0, (stage1 - n1p) * _D)))
    h1 = _tc_mlp(self1, mean1, W_agg1, b_agg1, W_enc1, b_enc1)

    nidx2 = _pad_rows(neigh_idx_l1.astype(i32), n2p).reshape(-1)
    w2 = _pad_rows(neigh_w_l1, n2p).reshape(-1)
    sp2 = _pad_rows(nodes_l2_pos.astype(i32), n2p)
    self2, mean2 = _make_sc_pool(n2p, bw_2, bw_2)(h1, h1, nidx2, sp2, w2)
    h2 = _tc_mlp(self2, mean2, W_agg2, b_agg2, W_enc2, b_enc2)
    return h2[:n2]


# speedup vs baseline: 1.7506x; 1.0249x over previous
import jax
import jax.numpy as jnp
from jax import lax
from jax.experimental import pallas as pl
from jax.experimental.pallas import tpu as pltpu
from jax.experimental.pallas import tpu_sc as plsc

_NC = 2
_NS = 16
_NW = _NC * _NS
_D = 16
_F = 128
_C = 8
_NBUF = 4
_SC_PARAMS = pltpu.CompilerParams(needs_layout_passes=False)

_GATHER_DNUMS = lax.GatherDimensionNumbers(
    offset_dims=(), collapsed_slice_dims=(0,), start_index_map=(0,))


def _lane_gather(vec, idx):
    return lax.gather(vec, idx[:, None], _GATHER_DNUMS, slice_sizes=(1,),
                      mode=lax.GatherScatterMode.PROMISE_IN_BOUNDS)


def _make_sc_compose(n_rows_pad, bw, n_ids):
    mesh = plsc.VectorSubcoreMesh(core_axis_name="c", subcore_axis_name="s")
    bwp = -(-bw // 128) * 128
    scratch = [
        pltpu.VMEM((n_ids,), jnp.int32),
        pltpu.VMEM((bwp,), jnp.int32),
    ]
    out_type = jax.ShapeDtypeStruct((n_rows_pad,), jnp.int32)

    def body(selfpos, ids_hbm, cself_out, ids_v, cself_v):
        wid = lax.axis_index("s") * _NC + lax.axis_index("c")
        base = wid * bw
        pltpu.sync_copy(ids_hbm, ids_v)
        pltpu.sync_copy(selfpos.at[pl.ds(base, bw)], cself_v.at[pl.ds(0, bw)])

        def comp_s(i, carry):
            v = cself_v[pl.ds(i * 16, 16)]
            cself_v[pl.ds(i * 16, 16)] = plsc.load_gather(ids_v, [v])
            return carry

        lax.fori_loop(0, bw // 16, comp_s, 0)
        pltpu.sync_copy(cself_v.at[pl.ds(0, bw)],
                        cself_out.at[pl.ds(base, bw)])

    return pl.kernel(body, out_type=out_type, mesh=mesh,
                     scratch_types=scratch, compiler_params=_SC_PARAMS,
                     name="sc_compose")


def _make_sc_pack(n_rows_pad, bw):
    nch = bw // 64
    assert nch >= 2
    mesh = plsc.VectorSubcoreMesh(core_axis_name="c", subcore_axis_name="s")
    scratch = [
        pltpu.VMEM((-(-bw // 128) * 128,), jnp.int32),
        pltpu.VMEM((2, 64, _F), jnp.float32),
        pltpu.VMEM((2, 64, _F // 2), jnp.int32),
        pltpu.SemaphoreType.DMA((2,)),
        pltpu.SemaphoreType.DMA((2,)),
    ]
    out_type = jax.ShapeDtypeStruct((n_rows_pad, _F // 2), jnp.int32)

    def body(table, ids, out, ids_v, rows_v, pk_v, isem, osem):
        wid = lax.axis_index("s") * _NC + lax.axis_index("c")
        base = wid * bw
        pltpu.sync_copy(ids.at[pl.ds(base, bw)], ids_v.at[pl.ds(0, bw)])

        def start_fetch(c, buf):
            pltpu.async_copy(table.at[ids_v.at[pl.ds(c * 64, 64)]],
                             rows_v.at[buf], isem.at[buf])

        def wait_out(c, buf):
            pltpu.make_async_copy(
                pk_v.at[buf], out.at[pl.ds(base + (c - 2) * 64, 64)],
                osem.at[buf]).wait()

        start_fetch(0, 0)

        def chunk(c, carry):
            buf = lax.rem(c, 2)

            @pl.when(c + 1 < nch)
            def _():
                nbuf = 1 - buf

                @pl.when(c + 1 >= 2)
                def _():
                    wait_out(c + 1, nbuf)

                start_fetch(c + 1, nbuf)

            pltpu.make_async_copy(table.at[ids_v.at[pl.ds(c * 64, 64)]],
                                  rows_v.at[buf], isem.at[buf]).wait()
            rbuf = rows_v.at[buf]
            pbuf = pk_v.at[buf]
            for rr in range(64):
                for m in range(_F // 32):
                    a = rbuf[rr, pl.ds(m * 16, 16)]
                    b = rbuf[rr, pl.ds(_F // 2 + m * 16, 16)]
                    pbuf[rr, pl.ds(m * 16, 16)] = plsc.bitcast(
                        plsc.pack(a, b, format=plsc.PackFormat.INTERLEAVED),
                        jnp.int32)
            pltpu.async_copy(pbuf, out.at[pl.ds(base + c * 64, 64)],
                             osem.at[buf])
            return carry

        lax.fori_loop(0, nch, chunk, 0)
        for b in range(2):
            wait_out(nch + b, lax.rem(nch + b, 2))

    return pl.kernel(body, out_type=out_type, mesh=mesh,
                     scratch_types=scratch,
                     compiler_params=pltpu.CompilerParams(
                         needs_layout_passes=False,
                         use_tc_tiling_on_sc=False),
                     name="sc_pack")


def _make_sc_pool(n_rows_pad, bw0, bw1, packed=False):
    nchunk0 = bw0 // _C
    nchunk1 = bw1 // _C
    bwm = max(bw0, bw1)
    assert min(nchunk0, nchunk1) >= _NBUF
    mesh = plsc.VectorSubcoreMesh(core_axis_name="c", subcore_axis_name="s")
    scratch = [
        pltpu.VMEM((bwm * _D,), jnp.int32),
        pltpu.VMEM((-(-bwm // 128) * 128,), jnp.int32),
        pltpu.VMEM((bwm * _D,), jnp.float32),
        pltpu.VMEM((_NBUF, _C * _D, _F // 2), jnp.int32) if packed
        else pltpu.VMEM((_NBUF, _C * _D, _F), jnp.float32),
        pltpu.VMEM((_NBUF, _C, _F), jnp.float32),
        pltpu.VMEM((_NBUF, _C, _F), jnp.float32),
        pltpu.SemaphoreType.DMA((_NBUF,)),
        pltpu.SemaphoreType.DMA((_NBUF,)),
    ]
    out_type = (
        jax.ShapeDtypeStruct((n_rows_pad, _F), jnp.float32),
        jax.ShapeDtypeStruct((n_rows_pad, _F), jnp.float32),
    )
    params = pltpu.CompilerParams(needs_layout_passes=False,
                                  use_tc_tiling_on_sc=False)

    def body(table_n, table_s, nidx, selfpos, w, self_out, mean_out,
             cidx_v, cself_v, w_v, rows_v, srows_v, nm_v, isem, osem):
        c = lax.axis_index("c")
        s = lax.axis_index("s")
        is0 = c == 0
        base = lax.select(is0, _NS * bw1 + s * bw0, s * bw1)
        nchunk = lax.select(is0, nchunk0, nchunk1)
        pltpu.sync_copy(nidx.at[pl.ds(base * _D, bwm * _D)], cidx_v)
        pltpu.sync_copy(selfpos.at[pl.ds(base, bwm)],
                        cself_v.at[pl.ds(0, bwm)])
        pltpu.sync_copy(w.at[pl.ds(base * _D, bwm * _D)], w_v)

        def start_fetch(c, buf):
            pltpu.async_copy(
                table_n.at[cidx_v.at[pl.ds(c * (_C * _D), _C * _D)]],
                rows_v.at[buf], isem.at[buf])
            pltpu.async_copy(
                table_s.at[cself_v.at[pl.ds(c * _C, _C)]],
                srows_v.at[buf], isem.at[buf])

        def wait_outputs(c, buf):
            pltpu.make_async_copy(
                srows_v.at[buf],
                self_out.at[pl.ds(base + (c - _NBUF) * _C, _C)],
                osem.at[buf]).wait()
            pltpu.make_async_copy(
                nm_v.at[buf],
                mean_out.at[pl.ds(base + (c - _NBUF) * _C, _C)],
                osem.at[buf]).wait()

        for b in range(_NBUF - 1):
            start_fetch(b, b)

        def chunk(c, carry):
            buf = lax.rem(c, _NBUF)
            nxt = c + _NBUF - 1

            @pl.when(nxt < nchunk)
            def _():
                nbuf = lax.rem(nxt, _NBUF)

                @pl.when(nxt >= _NBUF)
                def _():
                    wait_outputs(nxt, nbuf)

                start_fetch(nxt, nbuf)

            pltpu.make_async_copy(
                table_n.at[cidx_v.at[pl.ds(c * (_C * _D), _C * _D)]],
                rows_v.at[buf], isem.at[buf]).wait()
            pltpu.make_async_copy(
                table_s.at[cself_v.at[pl.ds(c * _C, _C)]],
                srows_v.at[buf], isem.at[buf]).wait()
            rb = rows_v.at[buf]
            nb = nm_v.at[buf]
            for b in range(_C):
                wv = w_v[pl.ds(c * (_C * _D) + b * _D, _D)]
                tot = _lane_gather(plsc.cumsum(wv),
                                   jnp.full((16,), _D - 1, jnp.int32))
                r = 1.0 / (tot + 1e-8)
                acc = [None] * (_F // 16)
                for j in range(_D):
                    wj = _lane_gather(wv, jnp.full((16,), j, jnp.int32))
                    if packed:
                        for k in range(_F // 32):
                            wd = rb[b * _D + j, pl.ds(k * 16, 16)]
                            lo = lax.bitcast_convert_type(
                                wd << 16, jnp.float32)
                            hi = lax.bitcast_convert_type(
                                wd & jnp.int32(-65536), jnp.float32)
                            xl = wj * lo
                            xh = wj * hi
                            kk = _F // 32 + k
                            acc[k] = xl if acc[k] is None else acc[k] + xl
                            acc[kk] = xh if acc[kk] is None else acc[kk] + xh
                    else:
                        for k in range(_F // 16):
                            x = wj * rb[b * _D + j, pl.ds(k * 16, 16)]
                            acc[k] = x if acc[k] is None else acc[k] + x
                for k in range(_F // 16):
                    nb[b, pl.ds(k * 16, 16)] = acc[k] * r
            pltpu.async_copy(srows_v.at[buf],
                             self_out.at[pl.ds(base + c * _C, _C)],
                             osem.at[buf])
            pltpu.async_copy(nb, mean_out.at[pl.ds(base + c * _C, _C)],
                             osem.at[buf])
            return carry

        lax.fori_loop(0, nchunk, chunk, 0, unroll=False)
        for b in range(_NBUF):
            wait_outputs(nchunk + b, lax.rem(nchunk + b, _NBUF))

    return pl.kernel(body, out_type=out_type, mesh=mesh,
                     scratch_types=scratch, compiler_params=params,
                     name="sc_pool")


def _tc_mlp_body(self_ref, nm_ref, wa_ref, ba_ref, ws_ref, wg_ref, be_ref,
                 out_ref):
    agg = jnp.dot(nm_ref[...], wa_ref[...],
                  preferred_element_type=jnp.float32) + ba_ref[...]
    agg = jnp.maximum(agg, 0.0)
    h = jnp.dot(self_ref[...], ws_ref[...],
                preferred_element_type=jnp.float32)
    h = h + jnp.dot(agg, wg_ref[...], preferred_element_type=jnp.float32)
    h = jnp.maximum(h + be_ref[...], 0.0)
    n = jnp.sqrt(jnp.sum(h * h, axis=1, keepdims=True)) + 1e-8
    out_ref[...] = h / n


def _tc_mlp(self_f, nm, Wa, ba, We, be, blk=256):
    n = self_f.shape[0]
    grid = (n // blk,)
    row_spec = pl.BlockSpec((blk, _F), lambda i: (i, 0))
    w_spec = pl.BlockSpec((_F, _F), lambda i: (0, 0))
    b_spec = pl.BlockSpec((1, _F), lambda i: (0, 0))
    return pl.pallas_call(
        _tc_mlp_body,
        grid=grid,
        in_specs=[row_spec, row_spec, w_spec, b_spec, w_spec, w_spec, b_spec],
        out_specs=row_spec,
        out_shape=jax.ShapeDtypeStruct((n, _F), jnp.float32),
    )(self_f, nm, Wa, ba.reshape(1, _F), We[:_F], We[_F:], be.reshape(1, _F))


def _pad_rows(x, n_pad):
    pad = [(0, n_pad - x.shape[0])] + [(0, 0)] * (x.ndim - 1)
    return jnp.pad(x, pad)


@jax.jit
def kernel(node_ids_l0, nodes_l1_pos, nodes_l2_pos, neigh_idx_l0, neigh_w_l0,
           neigh_idx_l1, neigh_w_l1, embedding_table,
           W_agg1, b_agg1, W_enc1, b_enc1, W_agg2, b_agg2, W_enc2, b_enc2):
    i32 = jnp.int32
    n1 = nodes_l1_pos.shape[0]
    n2 = nodes_l2_pos.shape[0]
    bwc = -(-n1 // (_NW * _C)) * _C
    n1p = bwc * _NW
    pair1 = n1p // _NS
    bw0_1 = max(_NBUF * _C, int(pair1 * 0.74) // _C * _C)
    bw1_1 = pair1 - bw0_1
    bw_2 = -(-n2 // (_NW * _C)) * _C
    n2p = bw_2 * _NW
    n0p = -(-node_ids_l0.shape[0] // 128) * 128

    nidx1 = _pad_rows(neigh_idx_l0.astype(i32), n1p).reshape(-1)
    w1 = _pad_rows(neigh_w_l0, n1p).reshape(-1)
    sp1 = _pad_rows(nodes_l1_pos.astype(i32), n1p)
    ids = node_ids_l0.astype(i32)
    bw_p = -(-ids.shape[0] // (_NW * 64)) * 64
    npk = bw_p * _NW
    packed0 = _make_sc_pack(npk, bw_p)(embedding_table, _pad_rows(ids, npk))
    cself1 = _make_sc_compose(n1p, bwc, n0p)(sp1, _pad_rows(ids, n0p))
    self1, mean1 = _make_sc_pool(n1p, bw0_1, bw1_1, packed=True)(
        packed0, embedding_table, nidx1, cself1, w1)
    h1 = _tc_mlp(self1, mean1, W_agg1, b_agg1, W_enc1, b_enc1)

    nidx2 = _pad_rows(neigh_idx_l1.astype(i32), n2p).reshape(-1)
    w2 = _pad_rows(neigh_w_l1, n2p).reshape(-1)
    sp2 = _pad_rows(nodes_l2_pos.astype(i32), n2p)
    self2, mean2 = _make_sc_pool(n2p, bw_2, bw_2)(h1, h1, nidx2, sp2, w2)
    h2 = _tc_mlp(self2, mean2, W_agg2, b_agg2, W_enc2, b_enc2)
    return h2[:n2]


# speedup vs baseline: 2.3616x; 1.3490x over previous
import jax
import jax.numpy as jnp
from jax import lax
from jax.experimental import pallas as pl
from jax.experimental.pallas import tpu as pltpu
from jax.experimental.pallas import tpu_sc as plsc

_NC = 2
_NS = 16
_NW = _NC * _NS
_D = 16
_F = 128
_C = 8
_NBUF = 4
_SC_PARAMS = pltpu.CompilerParams(needs_layout_passes=False)

_GATHER_DNUMS = lax.GatherDimensionNumbers(
    offset_dims=(), collapsed_slice_dims=(0,), start_index_map=(0,))


def _lane_gather(vec, idx):
    return lax.gather(vec, idx[:, None], _GATHER_DNUMS, slice_sizes=(1,),
                      mode=lax.GatherScatterMode.PROMISE_IN_BOUNDS)


def _make_sc_compose(n_rows_pad, bw, n_ids):
    mesh = plsc.VectorSubcoreMesh(core_axis_name="c", subcore_axis_name="s")
    bwp = -(-bw // 128) * 128
    scratch = [
        pltpu.VMEM((n_ids,), jnp.int32),
        pltpu.VMEM((bwp,), jnp.int32),
    ]
    out_type = jax.ShapeDtypeStruct((n_rows_pad,), jnp.int32)

    def body(selfpos, ids_hbm, cself_out, ids_v, cself_v):
        wid = lax.axis_index("s") * _NC + lax.axis_index("c")
        base = wid * bw
        pltpu.sync_copy(ids_hbm, ids_v)
        pltpu.sync_copy(selfpos.at[pl.ds(base, bw)], cself_v.at[pl.ds(0, bw)])

        def comp_s(i, carry):
            v = cself_v[pl.ds(i * 16, 16)]
            cself_v[pl.ds(i * 16, 16)] = plsc.load_gather(ids_v, [v])
            return carry

        lax.fori_loop(0, bw // 16, comp_s, 0)
        pltpu.sync_copy(cself_v.at[pl.ds(0, bw)],
                        cself_out.at[pl.ds(base, bw)])

    return pl.kernel(body, out_type=out_type, mesh=mesh,
                     scratch_types=scratch, compiler_params=_SC_PARAMS,
                     name="sc_compose")


def _make_sc_pack(n_rows_pad, bw0, bw1):
    nch0 = bw0 // 64
    nch1 = bw1 // 64
    bwm = max(bw0, bw1)
    assert min(nch0, nch1) >= 2
    mesh = plsc.VectorSubcoreMesh(core_axis_name="c", subcore_axis_name="s")
    scratch = [
        pltpu.VMEM((-(-bwm // 128) * 128,), jnp.int32),
        pltpu.VMEM((2, 64, _F), jnp.float32),
        pltpu.VMEM((2, 64, _F // 2), jnp.int32),
        pltpu.SemaphoreType.DMA((2,)),
        pltpu.SemaphoreType.DMA((2,)),
    ]
    out_type = jax.ShapeDtypeStruct((n_rows_pad, _F // 2), jnp.int32)

    def body(table, ids, out, ids_v, rows_v, pk_v, isem, osem):
        c = lax.axis_index("c")
        s = lax.axis_index("s")
        is0 = c == 0
        base = lax.select(is0, _NS * bw1 + s * bw0, s * bw1)
        nch = lax.select(is0, nch0, nch1)
        pltpu.sync_copy(ids.at[pl.ds(base, bwm)], ids_v.at[pl.ds(0, bwm)])

        def start_fetch(c, buf):
            pltpu.async_copy(table.at[ids_v.at[pl.ds(c * 64, 64)]],
                             rows_v.at[buf], isem.at[buf])

        def wait_out(c, buf):
            pltpu.make_async_copy(
                pk_v.at[buf], out.at[pl.ds(base + (c - 2) * 64, 64)],
                osem.at[buf]).wait()

        start_fetch(0, 0)

        def chunk(c, carry):
            buf = lax.rem(c, 2)

            @pl.when(c + 1 < nch)
            def _():
                nbuf = 1 - buf

                @pl.when(c + 1 >= 2)
                def _():
                    wait_out(c + 1, nbuf)

                start_fetch(c + 1, nbuf)

            pltpu.make_async_copy(table.at[ids_v.at[pl.ds(c * 64, 64)]],
                                  rows_v.at[buf], isem.at[buf]).wait()
            rbuf = rows_v.at[buf]
            pbuf = pk_v.at[buf]
            for rr in range(64):
                for m in range(_F // 32):
                    a = rbuf[rr, pl.ds(m * 16, 16)]
                    b = rbuf[rr, pl.ds(_F // 2 + m * 16, 16)]
                    pbuf[rr, pl.ds(m * 16, 16)] = plsc.bitcast(
                        plsc.pack(a, b, format=plsc.PackFormat.INTERLEAVED),
                        jnp.int32)
            pltpu.async_copy(pbuf, out.at[pl.ds(base + c * 64, 64)],
                             osem.at[buf])
            return carry

        lax.fori_loop(0, nch, chunk, 0, unroll=False)
        for b in range(2):
            wait_out(nch + b, lax.rem(nch + b, 2))

    return pl.kernel(body, out_type=out_type, mesh=mesh,
                     scratch_types=scratch,
                     compiler_params=pltpu.CompilerParams(
                         needs_layout_passes=False,
                         use_tc_tiling_on_sc=False),
                     name="sc_pack")


def _make_sc_pool(n_rows_pad, bw0, bw1, packed=False):
    nchunk0 = bw0 // _C
    nchunk1 = bw1 // _C
    bwm = max(bw0, bw1)
    assert min(nchunk0, nchunk1) >= _NBUF
    mesh = plsc.VectorSubcoreMesh(core_axis_name="c", subcore_axis_name="s")
    scratch = [
        pltpu.VMEM((bwm * _D,), jnp.int32),
        pltpu.VMEM((-(-bwm // 128) * 128,), jnp.int32),
        pltpu.VMEM((bwm * _D,), jnp.float32),
        pltpu.VMEM((_NBUF, _C * _D, _F // 2), jnp.int32) if packed
        else pltpu.VMEM((_NBUF, _C * _D, _F), jnp.float32),
        pltpu.VMEM((_NBUF, _C, _F), jnp.float32),
        pltpu.VMEM((_NBUF, _C, _F), jnp.float32),
        pltpu.SemaphoreType.DMA((_NBUF,)),
        pltpu.SemaphoreType.DMA((_NBUF,)),
    ]
    out_type = (
        jax.ShapeDtypeStruct((n_rows_pad, _F), jnp.float32),
        jax.ShapeDtypeStruct((n_rows_pad, _F), jnp.float32),
    )
    params = pltpu.CompilerParams(needs_layout_passes=False,
                                  use_tc_tiling_on_sc=False)

    def body(table_n, table_s, nidx, selfpos, w, self_out, mean_out,
             cidx_v, cself_v, w_v, rows_v, srows_v, nm_v, isem, osem):
        c = lax.axis_index("c")
        s = lax.axis_index("s")
        is0 = c == 0
        base = lax.select(is0, _NS * bw1 + s * bw0, s * bw1)
        nchunk = lax.select(is0, nchunk0, nchunk1)
        pltpu.sync_copy(nidx.at[pl.ds(base * _D, bwm * _D)], cidx_v)
        pltpu.sync_copy(selfpos.at[pl.ds(base, bwm)],
                        cself_v.at[pl.ds(0, bwm)])
        pltpu.sync_copy(w.at[pl.ds(base * _D, bwm * _D)], w_v)

        def start_fetch(c, buf):
            pltpu.async_copy(
                table_n.at[cidx_v.at[pl.ds(c * (_C * _D), _C * _D)]],
                rows_v.at[buf], isem.at[buf])
            pltpu.async_copy(
                table_s.at[cself_v.at[pl.ds(c * _C, _C)]],
                srows_v.at[buf], isem.at[buf])

        def wait_outputs(c, buf):
            pltpu.make_async_copy(
                srows_v.at[buf],
                self_out.at[pl.ds(base + (c - _NBUF) * _C, _C)],
                osem.at[buf]).wait()
            pltpu.make_async_copy(
                nm_v.at[buf],
                mean_out.at[pl.ds(base + (c - _NBUF) * _C, _C)],
                osem.at[buf]).wait()

        for b in range(_NBUF - 1):
            start_fetch(b, b)

        def chunk(c, carry):
            buf = lax.rem(c, _NBUF)
            nxt = c + _NBUF - 1

            @pl.when(nxt < nchunk)
            def _():
                nbuf = lax.rem(nxt, _NBUF)

                @pl.when(nxt >= _NBUF)
                def _():
                    wait_outputs(nxt, nbuf)

                start_fetch(nxt, nbuf)

            pltpu.make_async_copy(
                table_n.at[cidx_v.at[pl.ds(c * (_C * _D), _C * _D)]],
                rows_v.at[buf], isem.at[buf]).wait()
            pltpu.make_async_copy(
                table_s.at[cself_v.at[pl.ds(c * _C, _C)]],
                srows_v.at[buf], isem.at[buf]).wait()
            rb = rows_v.at[buf]
            nb = nm_v.at[buf]
            for b in range(_C):
                wv = w_v[pl.ds(c * (_C * _D) + b * _D, _D)]
                tot = _lane_gather(plsc.cumsum(wv),
                                   jnp.full((16,), _D - 1, jnp.int32))
                r = 1.0 / (tot + 1e-8)
                acc = [None] * (_F // 16)
                for j in range(_D):
                    wj = _lane_gather(wv, jnp.full((16,), j, jnp.int32))
                    if packed:
                        for k in range(_F // 32):
                            wd = rb[b * _D + j, pl.ds(k * 16, 16)]
                            lo = lax.bitcast_convert_type(
                                wd << 16, jnp.float32)
                            hi = lax.bitcast_convert_type(
                                wd & jnp.int32(-65536), jnp.float32)
                            xl = wj * lo
                            xh = wj * hi
                            kk = _F // 32 + k
                            acc[k] = xl if acc[k] is None else acc[k] + xl
                            acc[kk] = xh if acc[kk] is None else acc[kk] + xh
                    else:
                        for k in range(_F // 16):
                            x = wj * rb[b * _D + j, pl.ds(k * 16, 16)]
                            acc[k] = x if acc[k] is None else acc[k] + x
                for k in range(_F // 16):
                    nb[b, pl.ds(k * 16, 16)] = acc[k] * r
            pltpu.async_copy(srows_v.at[buf],
                             self_out.at[pl.ds(base + c * _C, _C)],
                             osem.at[buf])
            pltpu.async_copy(nb, mean_out.at[pl.ds(base + c * _C, _C)],
                             osem.at[buf])
            return carry

        lax.fori_loop(0, nchunk, chunk, 0, unroll=False)
        for b in range(_NBUF):
            wait_outputs(nchunk + b, lax.rem(nchunk + b, _NBUF))

    return pl.kernel(body, out_type=out_type, mesh=mesh,
                     scratch_types=scratch, compiler_params=params,
                     name="sc_pool")


def _tc_mlp_body(self_ref, nm_ref, wa_ref, ba_ref, ws_ref, wg_ref, be_ref,
                 out_ref):
    agg = jnp.dot(nm_ref[...], wa_ref[...],
                  preferred_element_type=jnp.float32) + ba_ref[...]
    agg = jnp.maximum(agg, 0.0)
    h = jnp.dot(self_ref[...], ws_ref[...],
                preferred_element_type=jnp.float32)
    h = h + jnp.dot(agg, wg_ref[...], preferred_element_type=jnp.float32)
    h = jnp.maximum(h + be_ref[...], 0.0)
    n = jnp.sqrt(jnp.sum(h * h, axis=1, keepdims=True)) + 1e-8
    out_ref[...] = h / n


def _tc_mlp(self_f, nm, Wa, ba, We, be, blk=512):
    n = self_f.shape[0]
    grid = (n // blk,)
    row_spec = pl.BlockSpec((blk, _F), lambda i: (i, 0))
    w_spec = pl.BlockSpec((_F, _F), lambda i: (0, 0))
    b_spec = pl.BlockSpec((1, _F), lambda i: (0, 0))
    return pl.pallas_call(
        _tc_mlp_body,
        grid=grid,
        in_specs=[row_spec, row_spec, w_spec, b_spec, w_spec, w_spec, b_spec],
        out_specs=row_spec,
        out_shape=jax.ShapeDtypeStruct((n, _F), jnp.float32),
    )(self_f, nm, Wa, ba.reshape(1, _F), We[:_F], We[_F:], be.reshape(1, _F))


def _pad_rows(x, n_pad):
    pad = [(0, n_pad - x.shape[0])] + [(0, 0)] * (x.ndim - 1)
    return jnp.pad(x, pad)


@jax.jit
def kernel(node_ids_l0, nodes_l1_pos, nodes_l2_pos, neigh_idx_l0, neigh_w_l0,
           neigh_idx_l1, neigh_w_l1, embedding_table,
           W_agg1, b_agg1, W_enc1, b_enc1, W_agg2, b_agg2, W_enc2, b_enc2):
    i32 = jnp.int32
    n1 = nodes_l1_pos.shape[0]
    n2 = nodes_l2_pos.shape[0]
    bwc = -(-n1 // (_NW * _C)) * _C
    n1p = bwc * _NW
    pair1 = n1p // _NS
    bw0_1 = max(_NBUF * _C, int(pair1 * 0.5) // _C * _C)
    bw1_1 = pair1 - bw0_1
    bw_2 = -(-n2 // (_NW * _C)) * _C
    n2p = bw_2 * _NW
    n0p = -(-node_ids_l0.shape[0] // 128) * 128

    nidx1 = _pad_rows(neigh_idx_l0.astype(i32), n1p).reshape(-1)
    w1 = _pad_rows(neigh_w_l0, n1p).reshape(-1)
    sp1 = _pad_rows(nodes_l1_pos.astype(i32), n1p)
    ids = node_ids_l0.astype(i32)
    pairp = -(-ids.shape[0] // (_NS * 64)) * 64
    npk = pairp * _NS
    bw0_p = max(128, int(pairp * 0.67) // 64 * 64)
    bw1_p = pairp - bw0_p
    packed0 = _make_sc_pack(npk, bw0_p, bw1_p)(
        embedding_table, _pad_rows(ids, npk))
    cself1 = _make_sc_compose(n1p, bwc, n0p)(sp1, _pad_rows(ids, n0p))
    self1, mean1 = _make_sc_pool(n1p, bw0_1, bw1_1, packed=True)(
        packed0, embedding_table, nidx1, cself1, w1)
    h1 = _tc_mlp(self1, mean1, W_agg1, b_agg1, W_enc1, b_enc1)

    nidx2 = _pad_rows(neigh_idx_l1.astype(i32), n2p).reshape(-1)
    w2 = _pad_rows(neigh_w_l1, n2p).reshape(-1)
    sp2 = _pad_rows(nodes_l2_pos.astype(i32), n2p)
    self2, mean2 = _make_sc_pool(n2p, bw_2, bw_2)(h1, h1, nidx2, sp2, w2)
    h2 = _tc_mlp(self2, mean2, W_agg2, b_agg2, W_enc2, b_enc2)
    return h2[:n2]


# speedup vs baseline: 2.3751x; 1.0057x over previous
import jax
import jax.numpy as jnp
from jax import lax
from jax.experimental import pallas as pl
from jax.experimental.pallas import tpu as pltpu
from jax.experimental.pallas import tpu_sc as plsc

_NC = 2
_NS = 16
_NW = _NC * _NS
_D = 16
_F = 128
_C = 8
_NBUF = 4
_SC_PARAMS = pltpu.CompilerParams(needs_layout_passes=False)

_GATHER_DNUMS = lax.GatherDimensionNumbers(
    offset_dims=(), collapsed_slice_dims=(0,), start_index_map=(0,))


def _lane_gather(vec, idx):
    return lax.gather(vec, idx[:, None], _GATHER_DNUMS, slice_sizes=(1,),
                      mode=lax.GatherScatterMode.PROMISE_IN_BOUNDS)


def _make_sc_compose(n_rows_pad, bw, n_ids):
    mesh = plsc.VectorSubcoreMesh(core_axis_name="c", subcore_axis_name="s")
    bwp = -(-bw // 128) * 128
    scratch = [
        pltpu.VMEM((n_ids,), jnp.int32),
        pltpu.VMEM((bwp,), jnp.int32),
    ]
    out_type = jax.ShapeDtypeStruct((n_rows_pad,), jnp.int32)

    def body(selfpos, ids_hbm, cself_out, ids_v, cself_v):
        wid = lax.axis_index("s") * _NC + lax.axis_index("c")
        base = wid * bw
        pltpu.sync_copy(ids_hbm, ids_v)
        pltpu.sync_copy(selfpos.at[pl.ds(base, bw)], cself_v.at[pl.ds(0, bw)])

        def comp_s(i, carry):
            v = cself_v[pl.ds(i * 16, 16)]
            cself_v[pl.ds(i * 16, 16)] = plsc.load_gather(ids_v, [v])
            return carry

        lax.fori_loop(0, bw // 16, comp_s, 0)
        pltpu.sync_copy(cself_v.at[pl.ds(0, bw)],
                        cself_out.at[pl.ds(base, bw)])

    return pl.kernel(body, out_type=out_type, mesh=mesh,
                     scratch_types=scratch, compiler_params=_SC_PARAMS,
                     name="sc_compose")


def _make_sc_pack(n_rows_pad, bw0, bw1):
    nch0 = bw0 // 64
    nch1 = bw1 // 64
    bwm = max(bw0, bw1)
    assert min(nch0, nch1) >= 2
    mesh = plsc.VectorSubcoreMesh(core_axis_name="c", subcore_axis_name="s")
    scratch = [
        pltpu.VMEM((-(-bwm // 128) * 128,), jnp.int32),
        pltpu.VMEM((2, 64, _F), jnp.float32),
        pltpu.VMEM((2, 64, _F // 2), jnp.int32),
        pltpu.SemaphoreType.DMA((2,)),
        pltpu.SemaphoreType.DMA((2,)),
    ]
    out_type = jax.ShapeDtypeStruct((n_rows_pad, _F // 2), jnp.int32)

    def body(table, ids, out, ids_v, rows_v, pk_v, isem, osem):
        c = lax.axis_index("c")
        s = lax.axis_index("s")
        is0 = c == 0
        base = lax.select(is0, _NS * bw1 + s * bw0, s * bw1)
        nch = lax.select(is0, nch0, nch1)
        pltpu.sync_copy(ids.at[pl.ds(base, bwm)], ids_v.at[pl.ds(0, bwm)])

        def start_fetch(c, buf):
            pltpu.async_copy(table.at[ids_v.at[pl.ds(c * 64, 64)]],
                             rows_v.at[buf], isem.at[buf])

        def wait_out(c, buf):
            pltpu.make_async_copy(
                pk_v.at[buf], out.at[pl.ds(base + (c - 2) * 64, 64)],
                osem.at[buf]).wait()

        start_fetch(0, 0)

        def chunk(c, carry):
            buf = lax.rem(c, 2)

            @pl.when(c + 1 < nch)
            def _():
                nbuf = 1 - buf

                @pl.when(c + 1 >= 2)
                def _():
                    wait_out(c + 1, nbuf)

                start_fetch(c + 1, nbuf)

            pltpu.make_async_copy(table.at[ids_v.at[pl.ds(c * 64, 64)]],
                                  rows_v.at[buf], isem.at[buf]).wait()
            rbuf = rows_v.at[buf]
            pbuf = pk_v.at[buf]
            for rr in range(64):
                for m in range(_F // 32):
                    a = rbuf[rr, pl.ds(m * 16, 16)]
                    b = rbuf[rr, pl.ds(_F // 2 + m * 16, 16)]
                    pbuf[rr, pl.ds(m * 16, 16)] = plsc.bitcast(
                        plsc.pack(a, b, format=plsc.PackFormat.INTERLEAVED),
                        jnp.int32)
            pltpu.async_copy(pbuf, out.at[pl.ds(base + c * 64, 64)],
                             osem.at[buf])
            return carry

        lax.fori_loop(0, nch, chunk, 0, unroll=False)
        for b in range(2):
            wait_out(nch + b, lax.rem(nch + b, 2))

    return pl.kernel(body, out_type=out_type, mesh=mesh,
                     scratch_types=scratch,
                     compiler_params=pltpu.CompilerParams(
                         needs_layout_passes=False,
                         use_tc_tiling_on_sc=False),
                     name="sc_pack")


def _make_sc_pool(n_rows_pad, bw0, bw1, packed=False):
    nchunk0 = bw0 // _C
    nchunk1 = bw1 // _C
    bwm = max(bw0, bw1)
    assert min(nchunk0, nchunk1) >= _NBUF
    mesh = plsc.VectorSubcoreMesh(core_axis_name="c", subcore_axis_name="s")
    scratch = [
        pltpu.VMEM((bwm * _D,), jnp.int32),
        pltpu.VMEM((-(-bwm // 128) * 128,), jnp.int32),
        pltpu.VMEM((bwm * _D,), jnp.float32),
        pltpu.VMEM((_NBUF, _C * _D, _F // 2), jnp.int32) if packed
        else pltpu.VMEM((_NBUF, _C * _D, _F), jnp.float32),
        pltpu.VMEM((_NBUF, _C, _F), jnp.float32),
        pltpu.VMEM((_NBUF, _C, _F), jnp.float32),
        pltpu.SemaphoreType.DMA((_NBUF,)),
        pltpu.SemaphoreType.DMA((_NBUF,)),
    ]
    out_type = (
        jax.ShapeDtypeStruct((n_rows_pad, _F), jnp.float32),
        jax.ShapeDtypeStruct((n_rows_pad, _F), jnp.float32),
    )
    params = pltpu.CompilerParams(needs_layout_passes=False,
                                  use_tc_tiling_on_sc=False)

    def body(table_n, table_s, nidx, selfpos, w, self_out, mean_out,
             cidx_v, cself_v, w_v, rows_v, srows_v, nm_v, isem, osem):
        c = lax.axis_index("c")
        s = lax.axis_index("s")
        is0 = c == 0
        base = lax.select(is0, _NS * bw1 + s * bw0, s * bw1)
        nchunk = lax.select(is0, nchunk0, nchunk1)
        pltpu.sync_copy(nidx.at[pl.ds(base * _D, bwm * _D)], cidx_v)
        pltpu.sync_copy(selfpos.at[pl.ds(base, bwm)],
                        cself_v.at[pl.ds(0, bwm)])
        pltpu.sync_copy(w.at[pl.ds(base * _D, bwm * _D)], w_v)

        def start_fetch(c, buf):
            pltpu.async_copy(
                table_n.at[cidx_v.at[pl.ds(c * (_C * _D), _C * _D)]],
                rows_v.at[buf], isem.at[buf])
            pltpu.async_copy(
                table_s.at[cself_v.at[pl.ds(c * _C, _C)]],
                srows_v.at[buf], isem.at[buf])

        def wait_outputs(c, buf):
            pltpu.make_async_copy(
                srows_v.at[buf],
                self_out.at[pl.ds(base + (c - _NBUF) * _C, _C)],
                osem.at[buf]).wait()
            pltpu.make_async_copy(
                nm_v.at[buf],
                mean_out.at[pl.ds(base + (c - _NBUF) * _C, _C)],
                osem.at[buf]).wait()

        for b in range(_NBUF - 1):
            start_fetch(b, b)

        def chunk(c, carry):
            buf = lax.rem(c, _NBUF)
            nxt = c + _NBUF - 1

            @pl.when(nxt < nchunk)
            def _():
                nbuf = lax.rem(nxt, _NBUF)

                @pl.when(nxt >= _NBUF)
                def _():
                    wait_outputs(nxt, nbuf)

                start_fetch(nxt, nbuf)

            pltpu.make_async_copy(
                table_n.at[cidx_v.at[pl.ds(c * (_C * _D), _C * _D)]],
                rows_v.at[buf], isem.at[buf]).wait()
            pltpu.make_async_copy(
                table_s.at[cself_v.at[pl.ds(c * _C, _C)]],
                srows_v.at[buf], isem.at[buf]).wait()
            rb = rows_v.at[buf]
            nb = nm_v.at[buf]
            for b in range(_C):
                wv = w_v[pl.ds(c * (_C * _D) + b * _D, _D)]
                tot = _lane_gather(plsc.cumsum(wv),
                                   jnp.full((16,), _D - 1, jnp.int32))
                r = 1.0 / (tot + 1e-8)
                acc = [None] * (_F // 16)
                for j in range(_D):
                    wj = _lane_gather(wv, jnp.full((16,), j, jnp.int32))
                    if packed:
                        for k in range(_F // 32):
                            wd = rb[b * _D + j, pl.ds(k * 16, 16)]
                            lo = lax.bitcast_convert_type(
                                wd << 16, jnp.float32)
                            hi = lax.bitcast_convert_type(
                                wd & jnp.int32(-65536), jnp.float32)
                            xl = wj * lo
                            xh = wj * hi
                            kk = _F // 32 + k
                            acc[k] = xl if acc[k] is None else acc[k] + xl
                            acc[kk] = xh if acc[kk] is None else acc[kk] + xh
                    else:
                        for k in range(_F // 16):
                            x = wj * rb[b * _D + j, pl.ds(k * 16, 16)]
                            acc[k] = x if acc[k] is None else acc[k] + x
                for k in range(_F // 16):
                    nb[b, pl.ds(k * 16, 16)] = acc[k] * r
            pltpu.async_copy(srows_v.at[buf],
                             self_out.at[pl.ds(base + c * _C, _C)],
                             osem.at[buf])
            pltpu.async_copy(nb, mean_out.at[pl.ds(base + c * _C, _C)],
                             osem.at[buf])
            return carry

        lax.fori_loop(0, nchunk, chunk, 0, unroll=False)
        for b in range(_NBUF):
            wait_outputs(nchunk + b, lax.rem(nchunk + b, _NBUF))

    return pl.kernel(body, out_type=out_type, mesh=mesh,
                     scratch_types=scratch, compiler_params=params,
                     name="sc_pool")


def _tc_mlp_body(self_ref, nm_ref, wa_ref, ba_ref, ws_ref, wg_ref, be_ref,
                 out_ref):
    agg = jnp.dot(nm_ref[...], wa_ref[...],
                  preferred_element_type=jnp.float32) + ba_ref[...]
    agg = jnp.maximum(agg, 0.0)
    h = jnp.dot(self_ref[...], ws_ref[...],
                preferred_element_type=jnp.float32)
    h = h + jnp.dot(agg, wg_ref[...], preferred_element_type=jnp.float32)
    h = jnp.maximum(h + be_ref[...], 0.0)
    n = jnp.sqrt(jnp.sum(h * h, axis=1, keepdims=True)) + 1e-8
    out_ref[...] = h / n


def _tc_mlp(self_f, nm, Wa, ba, We, be, blk=512):
    n = self_f.shape[0]
    grid = (n // blk,)
    row_spec = pl.BlockSpec((blk, _F), lambda i: (i, 0))
    w_spec = pl.BlockSpec((_F, _F), lambda i: (0, 0))
    b_spec = pl.BlockSpec((1, _F), lambda i: (0, 0))
    return pl.pallas_call(
        _tc_mlp_body,
        grid=grid,
        in_specs=[row_spec, row_spec, w_spec, b_spec, w_spec, w_spec, b_spec],
        out_specs=row_spec,
        out_shape=jax.ShapeDtypeStruct((n, _F), jnp.float32),
    )(self_f, nm, Wa, ba.reshape(1, _F), We[:_F], We[_F:], be.reshape(1, _F))


def _pad_rows(x, n_pad):
    pad = [(0, n_pad - x.shape[0])] + [(0, 0)] * (x.ndim - 1)
    return jnp.pad(x, pad)


@jax.jit
def kernel(node_ids_l0, nodes_l1_pos, nodes_l2_pos, neigh_idx_l0, neigh_w_l0,
           neigh_idx_l1, neigh_w_l1, embedding_table,
           W_agg1, b_agg1, W_enc1, b_enc1, W_agg2, b_agg2, W_enc2, b_enc2):
    i32 = jnp.int32
    n1 = nodes_l1_pos.shape[0]
    n2 = nodes_l2_pos.shape[0]
    bwc = -(-n1 // (_NW * _C)) * _C
    n1p = bwc * _NW
    pair1 = n1p // _NS
    bw0_1 = max(_NBUF * _C, int(pair1 * 0.40) // _C * _C)
    bw1_1 = pair1 - bw0_1
    bw_2 = -(-n2 // (_NW * _C)) * _C
    n2p = bw_2 * _NW
    n0p = -(-node_ids_l0.shape[0] // 128) * 128

    nidx1 = _pad_rows(neigh_idx_l0.astype(i32), n1p).reshape(-1)
    w1 = _pad_rows(neigh_w_l0, n1p).reshape(-1)
    sp1 = _pad_rows(nodes_l1_pos.astype(i32), n1p)
    ids = node_ids_l0.astype(i32)
    pairp = -(-ids.shape[0] // (_NS * 64)) * 64
    npk = pairp * _NS
    bw0_p = max(128, int(pairp * 0.47) // 64 * 64)
    bw1_p = pairp - bw0_p
    packed0 = _make_sc_pack(npk, bw0_p, bw1_p)(
        embedding_table, _pad_rows(ids, npk))
    cself1 = _make_sc_compose(n1p, bwc, n0p)(sp1, _pad_rows(ids, n0p))
    self1, mean1 = _make_sc_pool(n1p, bw0_1, bw1_1, packed=True)(
        packed0, embedding_table, nidx1, cself1, w1)
    h1 = _tc_mlp(self1, mean1, W_agg1, b_agg1, W_enc1, b_enc1)

    nidx2 = _pad_rows(neigh_idx_l1.astype(i32), n2p).reshape(-1)
    w2 = _pad_rows(neigh_w_l1, n2p).reshape(-1)
    sp2 = _pad_rows(nodes_l2_pos.astype(i32), n2p)
    self2, mean2 = _make_sc_pool(n2p, bw_2, bw_2)(h1, h1, nidx2, sp2, w2)
    h2 = _tc_mlp(self2, mean2, W_agg2, b_agg2, W_enc2, b_enc2)
    return h2[:n2]
